# Initial kernel scaffold; baseline (speedup 1.0000x reference)
import jax
import jax.numpy as jnp
from jax.experimental import pallas as pl


def kernel(x, edge_index, edge_attr, batch, W1, b1, W2, b2, Wlin, blin):
    raise NotImplementedError("write your pallas kernel here")


# speedup vs baseline: 22.2687x; 22.2687x over previous
import functools
import jax
import jax.numpy as jnp
from jax import lax
from jax.experimental import pallas as pl
from jax.experimental.pallas import tpu as pltpu
from jax.experimental.pallas import tpu_sc as plsc

NN = 100000
NP = 100352
EE = 3200000
EP = 3203072
NG = 128
F = 16

NC, NS = 2, 16
NW = NC * NS
EW = EP // NW
CH = 128
NCHUNK = EW // CH
RPT = NP // NS
NZCH = RPT // CH

_mesh = plsc.VectorSubcoreMesh(core_axis_name="c", subcore_axis_name="s")
_sc_params = pltpu.CompilerParams(use_tc_tiling_on_sc=False)


def _sc_deg_body(dst_hbm, out_hbm, dstv, onesv, zerov, acc, sem):
  c = lax.axis_index("c")
  s = lax.axis_index("s")
  w = s * NC + c
  for i in range(CH // 16):
    onesv[pl.ds(i * 16, 16)] = jnp.ones((16,), jnp.float32)
    zerov[pl.ds(i * 16, 16)] = jnp.zeros((16,), jnp.float32)

  def zloop(i, carry):
    pltpu.sync_copy(zerov, acc.at[pl.ds(s * RPT + i * CH, CH)])
    return carry
  lax.fori_loop(0, NZCH, zloop, 0)
  plsc.subcore_barrier()

  base = w * EW

  def eloop(i, carry):
    pltpu.sync_copy(dst_hbm.at[pl.ds(base + i * CH, CH)], dstv)
    pltpu.sync_copy(onesv, acc.at[dstv], add=True)
    return carry
  lax.fori_loop(0, NCHUNK, eloop, 0)
  plsc.subcore_barrier()

  pltpu.sync_copy(acc.at[pl.ds(s * RPT, RPT)],
                  out_hbm.at[pl.ds(c * NP + s * RPT, RPT)])


_sc_deg = functools.partial(
    pl.kernel,
    mesh=_mesh,
    out_type=jax.ShapeDtypeStruct((NC * NP,), jnp.float32),
    scratch_types=[
        pltpu.VMEM((CH,), jnp.int32),
        pltpu.VMEM((CH,), jnp.float32),
        pltpu.VMEM((CH,), jnp.float32),
        pltpu.VMEM_SHARED((NP,), jnp.float32),
        pltpu.SemaphoreType.DMA,
    ],
    compiler_params=_sc_params,
)(_sc_deg_body)


def _sc_agg_body(u_hbm, src_hbm, dst_hbm, zrows_hbm, out_hbm,
                 srcv, dstv, rows, zrows, acc, sem):
  c = lax.axis_index("c")
  s = lax.axis_index("s")
  w = s * NC + c
  pltpu.sync_copy(zrows_hbm, zrows)

  def zloop(i, carry):
    pltpu.sync_copy(zrows, acc.at[pl.ds(s * RPT + i * CH, CH)])
    return carry
  lax.fori_loop(0, NZCH, zloop, 0)
  plsc.subcore_barrier()

  base = w * EW

  def eloop(i, carry):
    pltpu.sync_copy(src_hbm.at[pl.ds(base + i * CH, CH)], srcv)
    pltpu.sync_copy(dst_hbm.at[pl.ds(base + i * CH, CH)], dstv)
    pltpu.async_copy(u_hbm.at[srcv], rows, sem).wait()
    pltpu.sync_copy(rows, acc.at[dstv], add=True)
    return carry
  lax.fori_loop(0, NCHUNK, eloop, 0)
  plsc.subcore_barrier()

  pltpu.sync_copy(acc.at[pl.ds(s * RPT, RPT)],
                  out_hbm.at[pl.ds(c * NP + s * RPT, RPT)])


_sc_agg = functools.partial(
    pl.kernel,
    mesh=_mesh,
    out_type=jax.ShapeDtypeStruct((NC * NP, F), jnp.float32),
    scratch_types=[
        pltpu.VMEM((CH,), jnp.int32),
        pltpu.VMEM((CH,), jnp.int32),
        pltpu.VMEM((CH, F), jnp.float32),
        pltpu.VMEM((CH, F), jnp.float32),
        pltpu.VMEM_SHARED((NP, F), jnp.float32),
        pltpu.SemaphoreType.DMA,
    ],
    compiler_params=_sc_params,
)(_sc_agg_body)


_R = 2048
_NBLK = NP // _R


def _dinv_block(degp_ref):
  return lax.rsqrt(degp_ref[0] + degp_ref[1] + 1.0)


def _tc_u1_body(degp_ref, x_ref, w1_ref, out_ref):
  dv = _dinv_block(degp_ref)
  h = jnp.dot(x_ref[...], w1_ref[...], preferred_element_type=jnp.float32)
  out_ref[...] = h * dv


def _tc_u1(degp2, xp, w1p):
  return pl.pallas_call(
      _tc_u1_body,
      grid=(_NBLK,),
      in_specs=[
          pl.BlockSpec((2, _R, 1), lambda i: (0, i, 0)),
          pl.BlockSpec((_R, 8), lambda i: (i, 0)),
          pl.BlockSpec((8, F), lambda i: (0, 0)),
      ],
      out_specs=pl.BlockSpec((_R, F), lambda i: (i, 0)),
      out_shape=jax.ShapeDtypeStruct((NP, F), jnp.float32),
  )(degp2, xp, w1p)


def _tc_u2_body(degp_ref, sp_ref, u1_ref, w2_ref, b1_ref, out_ref):
  dv = _dinv_block(degp_ref)
  z1 = jax.nn.relu(dv * (sp_ref[0] + sp_ref[1] + u1_ref[...]) + b1_ref[...])
  h2 = jnp.dot(z1, w2_ref[...], preferred_element_type=jnp.float32)
  out_ref[...] = h2 * dv


def _tc_u2(degp2, s1p, u1, w2, b1):
  return pl.pallas_call(
      _tc_u2_body,
      grid=(_NBLK,),
      in_specs=[
          pl.BlockSpec((2, _R, 1), lambda i: (0, i, 0)),
          pl.BlockSpec((2, _R, F), lambda i: (0, i, 0)),
          pl.BlockSpec((_R, F), lambda i: (i, 0)),
          pl.BlockSpec((F, F), lambda i: (0, 0)),
          pl.BlockSpec((1, F), lambda i: (0, 0)),
      ],
      out_specs=pl.BlockSpec((_R, F), lambda i: (i, 0)),
      out_shape=jax.ShapeDtypeStruct((NP, F), jnp.float32),
  )(degp2, s1p, u1, w2, b1)


def _tc_pool_body(degp_ref, sp_ref, u2_ref, b2_ref, bf_ref, out_ref):
  i = pl.program_id(0)
  dv = _dinv_block(degp_ref)
  z2 = dv * (sp_ref[0] + sp_ref[1] + u2_ref[...]) + b2_ref[...]
  gid = lax.broadcasted_iota(jnp.int32, (_R, NG), 1)
  oh = jnp.where(bf_ref[...] == gid, 1.0, 0.0)
  part = lax.dot_general(oh, z2, (((0,), (0,)), ((), ())),
                         preferred_element_type=jnp.float32)

  @pl.when(i == 0)
  def _():
    out_ref[...] = jnp.zeros_like(out_ref)

  out_ref[...] += part


def _tc_pool(degp2, s2p, u2, b2, batchf):
  return pl.pallas_call(
      _tc_pool_body,
      grid=(_NBLK,),
      in_specs=[
          pl.BlockSpec((2, _R, 1), lambda i: (0, i, 0)),
          pl.BlockSpec((2, _R, F), lambda i: (0, i, 0)),
          pl.BlockSpec((_R, F), lambda i: (i, 0)),
          pl.BlockSpec((1, F), lambda i: (0, 0)),
          pl.BlockSpec((_R, 1), lambda i: (i, 0)),
      ],
      out_specs=pl.BlockSpec((NG, F), lambda i: (0, 0)),
      out_shape=jax.ShapeDtypeStruct((NG, F), jnp.float32),
  )(degp2, s2p, u2, b2, batchf)


def _tc_head_body(g_ref, wl_ref, bl_ref, out_ref):
  logits = jnp.dot(g_ref[...], wl_ref[...],
                   preferred_element_type=jnp.float32) + bl_ref[...]
  valid = lax.broadcasted_iota(jnp.int32, (NG, 8), 1) < 7
  masked = jnp.where(valid, logits, -jnp.inf)
  m = jnp.max(masked, axis=1, keepdims=True)
  e = jnp.where(valid, jnp.exp(logits - m), 0.0)
  lse = jnp.log(jnp.sum(e, axis=1, keepdims=True))
  out_ref[...] = logits - m - lse


def _tc_head(g, wlp, blp):
  return pl.pallas_call(
      _tc_head_body,
      out_shape=jax.ShapeDtypeStruct((NG, 8), jnp.float32),
  )(g, wlp, blp)


def kernel(x, edge_index, edge_attr, batch, W1, b1, W2, b2, Wlin, blin):
  f32 = jnp.float32
  pad_e = jnp.full((EP - EE,), NN, jnp.int32)
  srcp = jnp.concatenate([edge_index[0], pad_e])
  dstp = jnp.concatenate([edge_index[1], pad_e])
  xp = jnp.pad(x, ((0, NP - NN), (0, 8 - x.shape[1])))
  w1p = jnp.pad(W1, ((0, 8 - W1.shape[0]), (0, 0)))
  batchf = jnp.pad(batch, (0, NP - NN),
                   constant_values=1000).reshape(NP, 1)
  zrows = jnp.zeros((CH, F), f32)
  wlp = jnp.pad(Wlin, ((0, 0), (0, 1)))
  blp = jnp.pad(blin, (0, 1)).reshape(1, 8)
  b1r = b1.reshape(1, F)
  b2r = b2.reshape(1, F)

  degp = _sc_deg(dstp)
  degp2 = degp.reshape(NC, NP, 1)

  u1 = _tc_u1(degp2, xp, w1p)
  s1p = _sc_agg(u1, srcp, dstp, zrows).reshape(NC, NP, F)
  u2 = _tc_u2(degp2, s1p, u1, W2, b1r)
  s2p = _sc_agg(u2, srcp, dstp, zrows).reshape(NC, NP, F)
  g = _tc_pool(degp2, s2p, u2, b2r, batchf)
  out = _tc_head(g, wlp, blp)
  return out[:, :7]


# speedup vs baseline: 57.4340x; 2.5791x over previous
import functools
import jax
import jax.numpy as jnp
from jax import lax
from jax.experimental import pallas as pl
from jax.experimental.pallas import tpu as pltpu
from jax.experimental.pallas import tpu_sc as plsc

NN = 100000
NP = 100352
EE = 3200000
EP = 3211264
NG = 128
F = 16

NC, NS = 2, 16
NW = NC * NS
EW = EP // NW
CH = 128
NCHUNK = EW // CH
EPC = EP // CH
RPT = NP // NS
GK = 2
NGRP = NCHUNK // GK

_mesh = plsc.VectorSubcoreMesh(core_axis_name="c", subcore_axis_name="s")
_sc_params = pltpu.CompilerParams(use_tc_tiling_on_sc=False)


def _sc_deg_body(e3_hbm, zr1_hbm, out_hbm, ib0, ib1, onesv, acc, sm0, sm1):
  c = lax.axis_index("c")
  s = lax.axis_index("s")
  w = s * NC + c
  for i in range(CH // 16):
    onesv[pl.ds(i * 16, 16)] = jnp.ones((16,), jnp.float32)
  pltpu.sync_copy(zr1_hbm, acc.at[pl.ds(s * RPT, RPT)])
  plsc.subcore_barrier()

  row0 = w * NCHUNK
  ibs = (ib0, ib1)
  sms = (sm0, sm1)

  def fire(g, p):
    pltpu.sync_copy(e3_hbm.at[pl.ds(row0 + g * GK, GK)], ibs[p])
    for j in range(GK):
      pltpu.async_copy(onesv, acc.at[ibs[p].at[j, 1]], sms[p], add=True)

  def drain(p):
    for j in range(GK):
      pltpu.make_async_copy(onesv, acc.at[ibs[p].at[j, 1]], sms[p]).wait()

  def body(i, carry):
    for q in range(2):
      @pl.when(i >= 1)
      def _():
        drain(q)
      fire(2 * i + q, q)
    return carry
  lax.fori_loop(0, NGRP // 2, body, 0)
  drain(0)
  drain(1)
  plsc.subcore_barrier()

  pltpu.sync_copy(acc.at[pl.ds(s * RPT, RPT)],
                  out_hbm.at[pl.ds(c * NP + s * RPT, RPT)])


_sc_deg = functools.partial(
    pl.kernel,
    mesh=_mesh,
    out_type=jax.ShapeDtypeStruct((NC * NP,), jnp.float32),
    scratch_types=[
        pltpu.VMEM((GK, 2, CH), jnp.int32),
        pltpu.VMEM((GK, 2, CH), jnp.int32),
        pltpu.VMEM((CH,), jnp.float32),
        pltpu.VMEM_SHARED((NP,), jnp.float32),
        pltpu.SemaphoreType.DMA,
        pltpu.SemaphoreType.DMA,
    ],
    compiler_params=_sc_params,
)(_sc_deg_body)


def _sc_agg_body(u_hbm, e3_hbm, zr16_hbm, out_hbm,
                 ib0, ib1, ib2, ib3, rw0, rw1, rw2, rw3, acc,
                 sg0, sg1, sg2, sg3, ss0, ss1, ss2, ss3):
  c = lax.axis_index("c")
  s = lax.axis_index("s")
  w = s * NC + c
  pltpu.sync_copy(zr16_hbm, acc.at[pl.ds(s * RPT, RPT)])
  plsc.subcore_barrier()

  row0 = w * NCHUNK
  ibs = (ib0, ib1, ib2, ib3)
  rws = (rw0, rw1, rw2, rw3)
  sgs = (sg0, sg1, sg2, sg3)
  sss = (ss0, ss1, ss2, ss3)

  def fire(g, p):
    pltpu.sync_copy(e3_hbm.at[pl.ds(row0 + g * GK, GK)], ibs[p])
    for j in range(GK):
      pltpu.async_copy(u_hbm.at[ibs[p].at[j, 0]], rws[p].at[j], sgs[p])

  def gds(p):
    for j in range(GK):
      pltpu.make_async_copy(u_hbm.at[ibs[p].at[j, 0]], rws[p].at[j],
                            sgs[p]).wait()
    for j in range(GK):
      pltpu.async_copy(rws[p].at[j], acc.at[ibs[p].at[j, 1]], sss[p],
                       add=True)

  def sdrain(p):
    for j in range(GK):
      pltpu.make_async_copy(rws[p].at[j], acc.at[ibs[p].at[j, 1]],
                            sss[p]).wait()

  def body(i, carry):
    for q in range(4):
      @pl.when(i >= 1)
      def _():
        sdrain(q)
      fire(4 * i + q, q)
      if q == 0:
        @pl.when(i >= 1)
        def _():
          gds(3)
      else:
        gds(q - 1)
    return carry
  lax.fori_loop(0, NGRP // 4, body, 0)
  gds(3)
  for p in range(4):
    sdrain(p)
  plsc.subcore_barrier()

  pltpu.sync_copy(acc.at[pl.ds(s * RPT, RPT)],
                  out_hbm.at[pl.ds(c * NP + s * RPT, RPT)])


_sc_agg = functools.partial(
    pl.kernel,
    mesh=_mesh,
    out_type=jax.ShapeDtypeStruct((NC * NP, F), jnp.float32),
    scratch_types=[
        pltpu.VMEM((GK, 2, CH), jnp.int32),
        pltpu.VMEM((GK, 2, CH), jnp.int32),
        pltpu.VMEM((GK, 2, CH), jnp.int32),
        pltpu.VMEM((GK, 2, CH), jnp.int32),
        pltpu.VMEM((GK, CH, F), jnp.float32),
        pltpu.VMEM((GK, CH, F), jnp.float32),
        pltpu.VMEM((GK, CH, F), jnp.float32),
        pltpu.VMEM((GK, CH, F), jnp.float32),
        pltpu.VMEM_SHARED((NP, F), jnp.float32),
        pltpu.SemaphoreType.DMA,
        pltpu.SemaphoreType.DMA,
        pltpu.SemaphoreType.DMA,
        pltpu.SemaphoreType.DMA,
        pltpu.SemaphoreType.DMA,
        pltpu.SemaphoreType.DMA,
        pltpu.SemaphoreType.DMA,
        pltpu.SemaphoreType.DMA,
    ],
    compiler_params=_sc_params,
)(_sc_agg_body)


_R = 2048
_NBLK = NP // _R


def _dinv_block(degp_ref):
  return lax.rsqrt(degp_ref[0] + degp_ref[1] + 1.0)


def _tc_u1_body(degp_ref, x_ref, w1_ref, out_ref):
  dv = _dinv_block(degp_ref)
  h = jnp.dot(x_ref[...], w1_ref[...], preferred_element_type=jnp.float32)
  out_ref[...] = h * dv


def _tc_u1(degp2, xp, w1p):
  return pl.pallas_call(
      _tc_u1_body,
      grid=(_NBLK,),
      in_specs=[
          pl.BlockSpec((2, _R, 1), lambda i: (0, i, 0)),
          pl.BlockSpec((_R, 8), lambda i: (i, 0)),
          pl.BlockSpec((8, F), lambda i: (0, 0)),
      ],
      out_specs=pl.BlockSpec((_R, F), lambda i: (i, 0)),
      out_shape=jax.ShapeDtypeStruct((NP, F), jnp.float32),
  )(degp2, xp, w1p)


def _tc_u2_body(degp_ref, sp_ref, u1_ref, w2_ref, b1_ref, out_ref):
  dv = _dinv_block(degp_ref)
  z1 = jax.nn.relu(dv * (sp_ref[0] + sp_ref[1] + u1_ref[...]) + b1_ref[...])
  h2 = jnp.dot(z1, w2_ref[...], preferred_element_type=jnp.float32)
  out_ref[...] = h2 * dv


def _tc_u2(degp2, s1p, u1, w2, b1):
  return pl.pallas_call(
      _tc_u2_body,
      grid=(_NBLK,),
      in_specs=[
          pl.BlockSpec((2, _R, 1), lambda i: (0, i, 0)),
          pl.BlockSpec((2, _R, F), lambda i: (0, i, 0)),
          pl.BlockSpec((_R, F), lambda i: (i, 0)),
          pl.BlockSpec((F, F), lambda i: (0, 0)),
          pl.BlockSpec((1, F), lambda i: (0, 0)),
      ],
      out_specs=pl.BlockSpec((_R, F), lambda i: (i, 0)),
      out_shape=jax.ShapeDtypeStruct((NP, F), jnp.float32),
  )(degp2, s1p, u1, w2, b1)


def _tc_pool_body(degp_ref, sp_ref, u2_ref, b2_ref, bf_ref, out_ref):
  i = pl.program_id(0)
  dv = _dinv_block(degp_ref)
  z2 = dv * (sp_ref[0] + sp_ref[1] + u2_ref[...]) + b2_ref[...]
  gid = lax.broadcasted_iota(jnp.int32, (_R, NG), 1)
  oh = jnp.where(bf_ref[...] == gid, 1.0, 0.0)
  part = lax.dot_general(oh, z2, (((0,), (0,)), ((), ())),
                         preferred_element_type=jnp.float32)

  @pl.when(i == 0)
  def _():
    out_ref[...] = jnp.zeros_like(out_ref)

  out_ref[...] += part


def _tc_pool(degp2, s2p, u2, b2, batchf):
  return pl.pallas_call(
      _tc_pool_body,
      grid=(_NBLK,),
      in_specs=[
          pl.BlockSpec((2, _R, 1), lambda i: (0, i, 0)),
          pl.BlockSpec((2, _R, F), lambda i: (0, i, 0)),
          pl.BlockSpec((_R, F), lambda i: (i, 0)),
          pl.BlockSpec((1, F), lambda i: (0, 0)),
          pl.BlockSpec((_R, 1), lambda i: (i, 0)),
      ],
      out_specs=pl.BlockSpec((NG, F), lambda i: (0, 0)),
      out_shape=jax.ShapeDtypeStruct((NG, F), jnp.float32),
  )(degp2, s2p, u2, b2, batchf)


def _tc_head_body(g_ref, wl_ref, bl_ref, out_ref):
  logits = jnp.dot(g_ref[...], wl_ref[...],
                   preferred_element_type=jnp.float32) + bl_ref[...]
  valid = lax.broadcasted_iota(jnp.int32, (NG, 8), 1) < 7
  masked = jnp.where(valid, logits, -jnp.inf)
  m = jnp.max(masked, axis=1, keepdims=True)
  e = jnp.where(valid, jnp.exp(logits - m), 0.0)
  lse = jnp.log(jnp.sum(e, axis=1, keepdims=True))
  out_ref[...] = logits - m - lse


def _tc_head(g, wlp, blp):
  return pl.pallas_call(
      _tc_head_body,
      out_shape=jax.ShapeDtypeStruct((NG, 8), jnp.float32),
  )(g, wlp, blp)


def kernel(x, edge_index, edge_attr, batch, W1, b1, W2, b2, Wlin, blin):
  f32 = jnp.float32
  ei = jnp.pad(edge_index, ((0, 0), (0, EP - EE)), constant_values=NN)
  e3 = jnp.swapaxes(ei.reshape(2, EPC, CH), 0, 1)
  xp = jnp.pad(x, ((0, NP - NN), (0, 8 - x.shape[1])))
  w1p = jnp.pad(W1, ((0, 8 - W1.shape[0]), (0, 0)))
  batchf = jnp.pad(batch, (0, NP - NN),
                   constant_values=1000).reshape(NP, 1)
  zr1 = jnp.zeros((RPT,), f32)
  zr16 = jnp.zeros((RPT, F), f32)
  wlp = jnp.pad(Wlin, ((0, 0), (0, 1)))
  blp = jnp.pad(blin, (0, 1)).reshape(1, 8)
  b1r = b1.reshape(1, F)
  b2r = b2.reshape(1, F)

  degp = _sc_deg(e3, zr1)
  degp2 = degp.reshape(NC, NP, 1)

  u1 = _tc_u1(degp2, xp, w1p)
  s1p = _sc_agg(u1, e3, zr16).reshape(NC, NP, F)
  u2 = _tc_u2(degp2, s1p, u1, W2, b1r)
  s2p = _sc_agg(u2, e3, zr16).reshape(NC, NP, F)
  g = _tc_pool(degp2, s2p, u2, b2r, batchf)
  out = _tc_head(g, wlp, blp)
  return out[:, :7]


# speedup vs baseline: 92.7143x; 1.6143x over previous
import functools
import jax
import jax.numpy as jnp
from jax import lax
from jax.experimental import pallas as pl
from jax.experimental.pallas import tpu as pltpu
from jax.experimental.pallas import tpu_sc as plsc

NN = 100000
NP = 100352
EE = 3200000
EP = 3211264
NG = 128
F = 16

NC, NS = 2, 16
NW = NC * NS
EW = EP // NW
CH = 128
NCHUNK = EW // CH
EPC = EP // CH
RPT = NP // NS
GK = 2
NGRP = NCHUNK // GK

_mesh = plsc.VectorSubcoreMesh(core_axis_name="c", subcore_axis_name="s")
_sc_params = pltpu.CompilerParams(use_tc_tiling_on_sc=False)


def _sc_deg_body(dstr_hbm, zr1_hbm, out_hbm,
                 db0, db1, db2, db3, onesv, acc,
                 si0, si1, si2, si3, ss0, ss1, ss2, ss3):
  c = lax.axis_index("c")
  s = lax.axis_index("s")
  w = s * NC + c
  for i in range(CH // 16):
    onesv[pl.ds(i * 16, 16)] = jnp.ones((16,), jnp.float32)
  pltpu.sync_copy(zr1_hbm, acc.at[pl.ds(s * RPT, RPT)])
  plsc.subcore_barrier()

  row0 = w * NCHUNK
  dbs = (db0, db1, db2, db3)
  sis = (si0, si1, si2, si3)
  sss = (ss0, ss1, ss2, ss3)

  def ifire(t, p):
    pltpu.async_copy(dstr_hbm.at[pl.ds(row0 + t * GK, GK)], dbs[p], sis[p])

  def sfire(p):
    pltpu.make_async_copy(dstr_hbm.at[pl.ds(0, GK)], dbs[p], sis[p]).wait()
    for j in range(GK):
      pltpu.async_copy(onesv, acc.at[dbs[p].at[j]], sss[p], add=True)

  def sdrain(p):
    for j in range(GK):
      pltpu.make_async_copy(onesv, acc.at[dbs[p].at[j]], sss[p]).wait()

  def body(i, carry):
    for q in range(4):
      @pl.when(i >= 1)
      def _():
        sdrain(q)
      ifire(4 * i + q, q)
      if q == 0:
        @pl.when(i >= 1)
        def _():
          sfire(3)
      else:
        sfire(q - 1)
    return carry
  lax.fori_loop(0, NGRP // 4, body, 0)
  sfire(3)
  for p in range(4):
    sdrain(p)
  plsc.subcore_barrier()

  pltpu.sync_copy(acc.at[pl.ds(s * RPT, RPT)],
                  out_hbm.at[pl.ds(c * NP + s * RPT, RPT)])


_sc_deg = functools.partial(
    pl.kernel,
    mesh=_mesh,
    out_type=jax.ShapeDtypeStruct((NC * NP,), jnp.float32),
    scratch_types=[
        pltpu.VMEM((GK, CH), jnp.int32),
        pltpu.VMEM((GK, CH), jnp.int32),
        pltpu.VMEM((GK, CH), jnp.int32),
        pltpu.VMEM((GK, CH), jnp.int32),
        pltpu.VMEM((CH,), jnp.float32),
        pltpu.VMEM_SHARED((NP,), jnp.float32),
    ] + [pltpu.SemaphoreType.DMA] * 8,
    compiler_params=_sc_params,
)(_sc_deg_body)


def _sc_agg_body(u_hbm, srcr_hbm, dstr_hbm, zr16_hbm, out_hbm,
                 sb0, sb1, sb2, sb3, db0, db1, db2, db3,
                 rw0, rw1, rw2, rw3, acc,
                 si0, si1, si2, si3, sg0, sg1, sg2, sg3,
                 ss0, ss1, ss2, ss3):
  c = lax.axis_index("c")
  s = lax.axis_index("s")
  w = s * NC + c
  pltpu.sync_copy(zr16_hbm, acc.at[pl.ds(s * RPT, RPT)])
  plsc.subcore_barrier()

  row0 = w * NCHUNK
  sbs = (sb0, sb1, sb2, sb3)
  dbs = (db0, db1, db2, db3)
  rws = (rw0, rw1, rw2, rw3)
  sis = (si0, si1, si2, si3)
  sgs = (sg0, sg1, sg2, sg3)
  sss = (ss0, ss1, ss2, ss3)

  def ifire(t, p):
    pltpu.async_copy(srcr_hbm.at[pl.ds(row0 + t * GK, GK)], sbs[p], sis[p])
    pltpu.async_copy(dstr_hbm.at[pl.ds(row0 + t * GK, GK)], dbs[p], sis[p])

  def gfire(p):
    pltpu.make_async_copy(srcr_hbm.at[pl.ds(0, GK)], sbs[p], sis[p]).wait()
    pltpu.make_async_copy(dstr_hbm.at[pl.ds(0, GK)], dbs[p], sis[p]).wait()
    for j in range(GK):
      pltpu.async_copy(u_hbm.at[sbs[p].at[j]], rws[p].at[j], sgs[p])

  def sfire(p):
    for j in range(GK):
      pltpu.make_async_copy(u_hbm.at[sbs[p].at[j]], rws[p].at[j],
                            sgs[p]).wait()
    for j in range(GK):
      pltpu.async_copy(rws[p].at[j], acc.at[dbs[p].at[j]], sss[p], add=True)

  def sdrain(p):
    for j in range(GK):
      pltpu.make_async_copy(rws[p].at[j], acc.at[dbs[p].at[j]], sss[p]).wait()

  def body(i, carry):
    for q in range(4):
      @pl.when(i >= 1)
      def _():
        sdrain(q)
      ifire(4 * i + q, q)
      if q == 0:
        @pl.when(i >= 1)
        def _():
          gfire(3)
      else:
        gfire(q - 1)
      if q <= 2:
        @pl.when(i >= 1)
        def _():
          sfire((q + 1) % 4)
      else:
        sfire(0)
    return carry
  lax.fori_loop(0, NGRP // 4, body, 0)
  gfire(3)
  sfire(1)
  sfire(2)
  sfire(3)
  for p in range(4):
    sdrain(p)
  plsc.subcore_barrier()

  pltpu.sync_copy(acc.at[pl.ds(s * RPT, RPT)],
                  out_hbm.at[pl.ds(c * NP + s * RPT, RPT)])


_sc_agg = functools.partial(
    pl.kernel,
    mesh=_mesh,
    out_type=jax.ShapeDtypeStruct((NC * NP, F), jnp.float32),
    scratch_types=[
        pltpu.VMEM((GK, CH), jnp.int32),
        pltpu.VMEM((GK, CH), jnp.int32),
        pltpu.VMEM((GK, CH), jnp.int32),
        pltpu.VMEM((GK, CH), jnp.int32),
        pltpu.VMEM((GK, CH), jnp.int32),
        pltpu.VMEM((GK, CH), jnp.int32),
        pltpu.VMEM((GK, CH), jnp.int32),
        pltpu.VMEM((GK, CH), jnp.int32),
        pltpu.VMEM((GK, CH, F), jnp.float32),
        pltpu.VMEM((GK, CH, F), jnp.float32),
        pltpu.VMEM((GK, CH, F), jnp.float32),
        pltpu.VMEM((GK, CH, F), jnp.float32),
        pltpu.VMEM_SHARED((NP, F), jnp.float32),
    ] + [pltpu.SemaphoreType.DMA] * 12,
    compiler_params=_sc_params,
)(_sc_agg_body)


_RB = 14
_R = NP // _RB
_RG = _R // 128
_DB = 784 // _RB


def _eyef():
  r = lax.broadcasted_iota(jnp.int32, (128, 128), 0)
  c = lax.broadcasted_iota(jnp.int32, (128, 128), 1)
  return jnp.where(r == c, 1.0, 0.0)


def _cols(lane_ref):
  eye = _eyef()
  parts = []
  for g in range(_RG):
    parts.append(lax.dot_general(eye, lane_ref[g:g + 1, :],
                                 (((1,), (1,)), ((), ())),
                                 preferred_element_type=jnp.float32))
  return jnp.concatenate(parts, axis=0)


def _tc_dinv_body(d0_ref, d1_ref, out_ref):
  out_ref[...] = lax.rsqrt(d0_ref[...] + d1_ref[...] + 1.0)


def _tc_dinv(degh):
  return pl.pallas_call(
      _tc_dinv_body,
      grid=(1,),
      in_specs=[
          pl.BlockSpec((784, 128), lambda i: (0, 0)),
          pl.BlockSpec((784, 128), lambda i: (1, 0)),
      ],
      out_specs=pl.BlockSpec((784, 128), lambda i: (0, 0)),
      out_shape=jax.ShapeDtypeStruct((784, 128), jnp.float32),
  )(degh, degh)


def _tc_u1_body(dv_ref, x_ref, w1_ref, out_ref):
  dvc = _cols(dv_ref)
  h = jnp.dot(x_ref[...], w1_ref[...], preferred_element_type=jnp.float32)
  out_ref[...] = h * dvc


def _tc_u1(dinv2, xp, w1p):
  return pl.pallas_call(
      _tc_u1_body,
      grid=(_RB,),
      in_specs=[
          pl.BlockSpec((_DB, 128), lambda i: (i, 0)),
          pl.BlockSpec((_R, 8), lambda i: (i, 0)),
          pl.BlockSpec((8, F), lambda i: (0, 0)),
      ],
      out_specs=pl.BlockSpec((_R, F), lambda i: (i, 0)),
      out_shape=jax.ShapeDtypeStruct((NP, F), jnp.float32),
  )(dinv2, xp, w1p)


def _tc_u2_body(dv_ref, s0_ref, s1_ref, u1_ref, w2_ref, b1_ref, out_ref):
  dvc = _cols(dv_ref)
  z1 = jax.nn.relu(dvc * (s0_ref[...] + s1_ref[...] + u1_ref[...])
                   + b1_ref[...])
  h2 = jnp.dot(z1, w2_ref[...], preferred_element_type=jnp.float32)
  out_ref[...] = h2 * dvc


def _tc_u2(dinv2, sf, u1, w2, b1):
  return pl.pallas_call(
      _tc_u2_body,
      grid=(_RB,),
      in_specs=[
          pl.BlockSpec((_DB, 128), lambda i: (i, 0)),
          pl.BlockSpec((_R, F), lambda i: (i, 0)),
          pl.BlockSpec((_R, F), lambda i: (i + _RB, 0)),
          pl.BlockSpec((_R, F), lambda i: (i, 0)),
          pl.BlockSpec((F, F), lambda i: (0, 0)),
          pl.BlockSpec((1, F), lambda i: (0, 0)),
      ],
      out_specs=pl.BlockSpec((_R, F), lambda i: (i, 0)),
      out_shape=jax.ShapeDtypeStruct((NP, F), jnp.float32),
  )(dinv2, sf, sf, u1, w2, b1)


def _tc_pool_body(dv_ref, s0_ref, s1_ref, u2_ref, b2_ref, bf_ref, out_ref):
  i = pl.program_id(0)
  dvc = _cols(dv_ref)
  z2 = dvc * (s0_ref[...] + s1_ref[...] + u2_ref[...]) + b2_ref[...]
  eye = _eyef()
  gidf = lax.broadcasted_iota(jnp.int32, (128, NG), 1).astype(jnp.float32)
  part = jnp.zeros((NG, F), jnp.float32)
  for g in range(_RG):
    bcol = lax.dot_general(eye, bf_ref[g:g + 1, :],
                           (((1,), (1,)), ((), ())),
                           preferred_element_type=jnp.float32)
    oh = jnp.where(bcol == gidf, 1.0, 0.0)
    part = part + lax.dot_general(oh, z2[g * 128:(g + 1) * 128, :],
                                  (((0,), (0,)), ((), ())),
                                  preferred_element_type=jnp.float32)

  @pl.when(i == 0)
  def _():
    out_ref[...] = jnp.zeros_like(out_ref)

  out_ref[...] += part


def _tc_pool(dinv2, sf, u2, b2, batchf):
  return pl.pallas_call(
      _tc_pool_body,
      grid=(_RB,),
      in_specs=[
          pl.BlockSpec((_DB, 128), lambda i: (i, 0)),
          pl.BlockSpec((_R, F), lambda i: (i, 0)),
          pl.BlockSpec((_R, F), lambda i: (i + _RB, 0)),
          pl.BlockSpec((_R, F), lambda i: (i, 0)),
          pl.BlockSpec((1, F), lambda i: (0, 0)),
          pl.BlockSpec((_DB, 128), lambda i: (i, 0)),
      ],
      out_specs=pl.BlockSpec((NG, F), lambda i: (0, 0)),
      out_shape=jax.ShapeDtypeStruct((NG, F), jnp.float32),
  )(dinv2, sf, sf, u2, b2, batchf)


def _tc_head_body(g_ref, wl_ref, bl_ref, out_ref):
  logits = jnp.dot(g_ref[...], wl_ref[...],
                   preferred_element_type=jnp.float32) + bl_ref[...]
  valid = lax.broadcasted_iota(jnp.int32, (NG, 8), 1) < 7
  masked = jnp.where(valid, logits, -jnp.inf)
  m = jnp.max(masked, axis=1, keepdims=True)
  e = jnp.where(valid, jnp.exp(logits - m), 0.0)
  lse = jnp.log(jnp.sum(e, axis=1, keepdims=True))
  out_ref[...] = logits - m - lse


def _tc_head(g, wlp, blp):
  return pl.pallas_call(
      _tc_head_body,
      out_shape=jax.ShapeDtypeStruct((NG, 8), jnp.float32),
  )(g, wlp, blp)


def kernel(x, edge_index, edge_attr, batch, W1, b1, W2, b2, Wlin, blin):
  f32 = jnp.float32
  pad_idx = NN + jnp.arange(EP - EE, dtype=jnp.int32) % (NP - NN)
  srcr = jnp.concatenate([edge_index[0], pad_idx]).reshape(EPC, CH)
  dstr = jnp.concatenate([edge_index[1], pad_idx]).reshape(EPC, CH)
  xp = jnp.pad(x, ((0, NP - NN), (0, 8 - x.shape[1])))
  w1p = jnp.pad(W1, ((0, 8 - W1.shape[0]), (0, 0)))
  batchf = jnp.pad(batch, (0, NP - NN),
                   constant_values=1000).astype(f32).reshape(784, 128)
  zr1 = jnp.zeros((RPT,), f32)
  zr16 = jnp.zeros((RPT, F), f32)
  wlp = jnp.pad(Wlin, ((0, 0), (0, 1)))
  blp = jnp.pad(blin, (0, 1)).reshape(1, 8)
  b1r = b1.reshape(1, F)
  b2r = b2.reshape(1, F)

  degp = _sc_deg(dstr, zr1)
  dinv2 = _tc_dinv(degp.reshape(NC * 784, 128))

  u1 = _tc_u1(dinv2, xp, w1p)
  s1f = _sc_agg(u1, srcr, dstr, zr16)
  u2 = _tc_u2(dinv2, s1f, u1, W2, b1r)
  s2f = _sc_agg(u2, srcr, dstr, zr16)
  g = _tc_pool(dinv2, s2f, u2, b2r, batchf)
  out = _tc_head(g, wlp, blp)
  return out[:, :7]


# speedup vs baseline: 122.3031x; 1.3191x over previous
import functools
import jax
import jax.numpy as jnp
from jax import lax
from jax.experimental import pallas as pl
from jax.experimental.pallas import tpu as pltpu
from jax.experimental.pallas import tpu_sc as plsc

NN = 100000
NP = 100352
EE = 3200000
EP = 3211264
NG = 128
F = 16

NC, NS = 2, 16
NW = NC * NS
EW = EP // NW
CH = 128
NCHUNK = EW // CH
RPT = NP // NS

_mesh = plsc.VectorSubcoreMesh(core_axis_name="c", subcore_axis_name="s")
_sc_params = pltpu.CompilerParams(use_tc_tiling_on_sc=False)


_NSL = 8


def _sc_deg_body(dst_hbm, zr16_hbm, on16_hbm, out_hbm,
                 db0, db1, db2, db3, db4, db5, db6, db7, ones16, acc,
                 *sems):
  c = lax.axis_index("c")
  s = lax.axis_index("s")
  w = s * NC + c
  pltpu.sync_copy(on16_hbm, ones16)
  pltpu.sync_copy(zr16_hbm, acc.at[pl.ds(s * RPT, RPT)])
  plsc.subcore_barrier()

  base = w * EW
  dbs = (db0, db1, db2, db3, db4, db5, db6, db7)
  sis = sems[:_NSL]
  sss = sems[_NSL:]

  def ifire(t, p):
    pltpu.async_copy(dst_hbm.at[pl.ds(base + t * CH, CH)], dbs[p], sis[p])

  def sfire(p):
    pltpu.make_async_copy(dst_hbm.at[pl.ds(0, CH)], dbs[p], sis[p]).wait()
    pltpu.async_copy(ones16, acc.at[dbs[p]], sss[p], add=True)

  def sdrain(p):
    pltpu.make_async_copy(ones16, acc.at[dbs[p]], sss[p]).wait()

  def body(i, carry):
    for q in range(_NSL):
      @pl.when(i >= 1)
      def _():
        sdrain(q)
      ifire(_NSL * i + q, q)
      if q >= 2:
        sfire(q - 2)
      else:
        @pl.when(i >= 1)
        def _():
          sfire((q + 6) % _NSL)
    return carry
  lax.fori_loop(0, NCHUNK // _NSL, body, 0)
  sfire(6)
  sfire(7)
  for p in range(_NSL):
    sdrain(p)
  plsc.subcore_barrier()

  pltpu.sync_copy(acc.at[pl.ds(s * RPT, RPT)],
                  out_hbm.at[pl.ds(c * NP + s * RPT, RPT)])


_sc_deg = functools.partial(
    pl.kernel,
    mesh=_mesh,
    out_type=jax.ShapeDtypeStruct((NC * NP, F), jnp.float32),
    scratch_types=[pltpu.VMEM((CH,), jnp.int32)] * 8 + [
        pltpu.VMEM((CH, F), jnp.float32),
        pltpu.VMEM_SHARED((NP, F), jnp.float32),
    ] + [pltpu.SemaphoreType.DMA] * 16,
    compiler_params=_sc_params,
)(_sc_deg_body)


def _sc_agg_body(u_hbm, src_hbm, dst_hbm, zr16_hbm, out_hbm,
                 sb0, sb1, sb2, sb3, sb4, sb5, sb6, sb7,
                 db0, db1, db2, db3, db4, db5, db6, db7,
                 rw0, rw1, rw2, rw3, rw4, rw5, rw6, rw7, acc,
                 *sems):
  c = lax.axis_index("c")
  s = lax.axis_index("s")
  w = s * NC + c
  pltpu.sync_copy(zr16_hbm, acc.at[pl.ds(s * RPT, RPT)])
  plsc.subcore_barrier()

  base = w * EW
  sbs = (sb0, sb1, sb2, sb3, sb4, sb5, sb6, sb7)
  dbs = (db0, db1, db2, db3, db4, db5, db6, db7)
  rws = (rw0, rw1, rw2, rw3, rw4, rw5, rw6, rw7)
  sis = sems[:_NSL]
  sgs = sems[_NSL:2 * _NSL]
  sss = sems[2 * _NSL:]

  def ifire(t, p):
    pltpu.async_copy(src_hbm.at[pl.ds(base + t * CH, CH)], sbs[p], sis[p])
    pltpu.async_copy(dst_hbm.at[pl.ds(base + t * CH, CH)], dbs[p], sis[p])

  def gfire(p):
    pltpu.make_async_copy(src_hbm.at[pl.ds(0, CH)], sbs[p], sis[p]).wait()
    pltpu.make_async_copy(dst_hbm.at[pl.ds(0, CH)], dbs[p], sis[p]).wait()
    pltpu.async_copy(u_hbm.at[sbs[p]], rws[p], sgs[p])

  def sfire(p):
    pltpu.make_async_copy(u_hbm.at[sbs[p]], rws[p], sgs[p]).wait()
    pltpu.async_copy(rws[p], acc.at[dbs[p]], sss[p], add=True)

  def sdrain(p):
    pltpu.make_async_copy(rws[p], acc.at[dbs[p]], sss[p]).wait()

  def body(i, carry):
    for q in range(_NSL):
      @pl.when(i >= 1)
      def _():
        sdrain(q)
      ifire(_NSL * i + q, q)
      if q >= 2:
        gfire(q - 2)
      else:
        @pl.when(i >= 1)
        def _():
          gfire((q + 6) % _NSL)
      if q >= 5:
        sfire(q - 5)
      else:
        @pl.when(i >= 1)
        def _():
          sfire((q + 3) % _NSL)
    return carry
  lax.fori_loop(0, NCHUNK // _NSL, body, 0)
  gfire(6)
  gfire(7)
  for p in (3, 4, 5, 6, 7):
    sfire(p)
  for p in range(_NSL):
    sdrain(p)
  plsc.subcore_barrier()

  pltpu.sync_copy(acc.at[pl.ds(s * RPT, RPT)],
                  out_hbm.at[pl.ds(c * NP + s * RPT, RPT)])


_sc_agg = functools.partial(
    pl.kernel,
    mesh=_mesh,
    out_type=jax.ShapeDtypeStruct((NC * NP, F), jnp.float32),
    scratch_types=[pltpu.VMEM((CH,), jnp.int32)] * 16 + [
        pltpu.VMEM((CH, F), jnp.float32)] * 8 + [
        pltpu.VMEM_SHARED((NP, F), jnp.float32),
    ] + [pltpu.SemaphoreType.DMA] * 24,
    compiler_params=_sc_params,
)(_sc_agg_body)


NR = NP * F // 128
_GB = 4
_R = NR // _GB


def _tc_dinv_body(d0_ref, d1_ref, out_ref):
  out_ref[...] = lax.rsqrt(d0_ref[...] + d1_ref[...] + 1.0)


def _tc_dinv(deg128):
  return pl.pallas_call(
      _tc_dinv_body,
      grid=(_GB,),
      in_specs=[
          pl.BlockSpec((_R, 128), lambda i: (i, 0)),
          pl.BlockSpec((_R, 128), lambda i: (i + _GB, 0)),
      ],
      out_specs=pl.BlockSpec((_R, 128), lambda i: (i, 0)),
      out_shape=jax.ShapeDtypeStruct((NR, 128), jnp.float32),
  )(deg128, deg128)


def _tc_u1_body(dv_ref, x_ref, w1_ref, out_ref):
  h = jnp.dot(x_ref[...], w1_ref[...], preferred_element_type=jnp.float32)
  out_ref[...] = h * dv_ref[...]


def _tc_u1(dinv128, x128, w1bd):
  return pl.pallas_call(
      _tc_u1_body,
      grid=(_GB,),
      in_specs=[
          pl.BlockSpec((_R, 128), lambda i: (i, 0)),
          pl.BlockSpec((_R, 128), lambda i: (i, 0)),
          pl.BlockSpec((128, 128), lambda i: (0, 0)),
      ],
      out_specs=pl.BlockSpec((_R, 128), lambda i: (i, 0)),
      out_shape=jax.ShapeDtypeStruct((NR, 128), jnp.float32),
  )(dinv128, x128, w1bd)


def _tc_u2_body(dv_ref, s0_ref, s1_ref, u1_ref, w2_ref, b1_ref, out_ref):
  z1 = jax.nn.relu(dv_ref[...] * (s0_ref[...] + s1_ref[...] + u1_ref[...])
                   + b1_ref[...])
  h2 = jnp.dot(z1, w2_ref[...], preferred_element_type=jnp.float32)
  out_ref[...] = h2 * dv_ref[...]


def _tc_u2(dinv128, s128, u1, w2bd, b1t):
  return pl.pallas_call(
      _tc_u2_body,
      grid=(_GB,),
      in_specs=[
          pl.BlockSpec((_R, 128), lambda i: (i, 0)),
          pl.BlockSpec((_R, 128), lambda i: (i, 0)),
          pl.BlockSpec((_R, 128), lambda i: (i + _GB, 0)),
          pl.BlockSpec((_R, 128), lambda i: (i, 0)),
          pl.BlockSpec((128, 128), lambda i: (0, 0)),
          pl.BlockSpec((1, 128), lambda i: (0, 0)),
      ],
      out_specs=pl.BlockSpec((_R, 128), lambda i: (i, 0)),
      out_shape=jax.ShapeDtypeStruct((NR, 128), jnp.float32),
  )(dinv128, s128, s128, u1, w2bd, b1t)


def _tc_pool_body(dv_ref, s0_ref, s1_ref, u2_ref, b2_ref, bf_ref, out_ref):
  i = pl.program_id(0)
  z2 = dv_ref[...] * (s0_ref[...] + s1_ref[...] + u2_ref[...]) + b2_ref[...]
  gidf = lax.broadcasted_iota(jnp.int32, (_R, NG), 1).astype(jnp.float32)
  part = jnp.zeros((NG, F), jnp.float32)
  for j in range(8):
    bcol = bf_ref[:, 16 * j:16 * j + 1]
    oh = jnp.where(bcol == gidf, 1.0, 0.0)
    part = part + lax.dot_general(oh, z2[:, 16 * j:16 * j + 16],
                                  (((0,), (0,)), ((), ())),
                                  preferred_element_type=jnp.float32)

  @pl.when(i == 0)
  def _():
    out_ref[...] = jnp.zeros_like(out_ref)

  out_ref[...] += part


def _tc_pool(dinv128, s128, u2, b2t, batch128):
  return pl.pallas_call(
      _tc_pool_body,
      grid=(_GB,),
      in_specs=[
          pl.BlockSpec((_R, 128), lambda i: (i, 0)),
          pl.BlockSpec((_R, 128), lambda i: (i, 0)),
          pl.BlockSpec((_R, 128), lambda i: (i + _GB, 0)),
          pl.BlockSpec((_R, 128), lambda i: (i, 0)),
          pl.BlockSpec((1, 128), lambda i: (0, 0)),
          pl.BlockSpec((_R, 128), lambda i: (i, 0)),
      ],
      out_specs=pl.BlockSpec((NG, F), lambda i: (0, 0)),
      out_shape=jax.ShapeDtypeStruct((NG, F), jnp.float32),
  )(dinv128, s128, s128, u2, b2t, batch128)


def _tc_head_body(g_ref, wl_ref, bl_ref, out_ref):
  logits = jnp.dot(g_ref[...], wl_ref[...],
                   preferred_element_type=jnp.float32) + bl_ref[...]
  valid = lax.broadcasted_iota(jnp.int32, (NG, 8), 1) < 7
  masked = jnp.where(valid, logits, -jnp.inf)
  m = jnp.max(masked, axis=1, keepdims=True)
  e = jnp.where(valid, jnp.exp(logits - m), 0.0)
  lse = jnp.log(jnp.sum(e, axis=1, keepdims=True))
  out_ref[...] = logits - m - lse


def _tc_head(g, wlp, blp):
  return pl.pallas_call(
      _tc_head_body,
      out_shape=jax.ShapeDtypeStruct((NG, 8), jnp.float32),
  )(g, wlp, blp)


def kernel(x, edge_index, edge_attr, batch, W1, b1, W2, b2, Wlin, blin):
  f32 = jnp.float32
  pad_idx = NN + jnp.arange(EP - EE, dtype=jnp.int32) % (NP - NN)
  srcr = jnp.concatenate([edge_index[0], pad_idx])
  dstr = jnp.concatenate([edge_index[1], pad_idx])
  x128 = jnp.pad(x, ((0, NP - NN), (0, F - x.shape[1]))).reshape(NR, 128)
  w1bd = jnp.kron(jnp.eye(8, dtype=f32),
                  jnp.pad(W1, ((0, F - W1.shape[0]), (0, 0))))
  w2bd = jnp.kron(jnp.eye(8, dtype=f32), W2)
  batch128 = jnp.repeat(
      jnp.pad(batch, (0, NP - NN), constant_values=1000), F
  ).astype(f32).reshape(NR, 128)
  zr16 = jnp.zeros((RPT, F), f32)
  on16 = jnp.ones((CH, F), f32)
  b1t = jnp.tile(b1, 8).reshape(1, 128)
  b2t = jnp.tile(b2, 8).reshape(1, 128)
  wlp = jnp.pad(Wlin, ((0, 0), (0, 1)))
  blp = jnp.pad(blin, (0, 1)).reshape(1, 8)

  degp16 = _sc_deg(dstr, zr16, on16)
  dinv128 = _tc_dinv(degp16.reshape(2 * NR, 128))

  u1 = _tc_u1(dinv128, x128, w1bd)
  s1f = _sc_agg(u1.reshape(NP, F), srcr, dstr, zr16)
  u2 = _tc_u2(dinv128, s1f.reshape(2 * NR, 128), u1, w2bd, b1t)
  s2f = _sc_agg(u2.reshape(NP, F), srcr, dstr, zr16)
  g = _tc_pool(dinv128, s2f.reshape(2 * NR, 128), u2, b2t, batch128)
  out = _tc_head(g, wlp, blp)
  return out[:, :7]
